# baseline (speedup 1.0000x reference)
import functools

import jax
import jax.numpy as jnp
from jax.experimental import pallas as pl
from jax.experimental.pallas import tpu as pltpu

EMBED_DIM = 256
IMG_H, IMG_W = 1024, 1024

_S0, _S1, _S2, _S3 = 6.278553964, -41.09111634, 77.90940339, -56.03846994
_C0, _C1, _C2, _C3, _C4 = (
    0.9999590208, -19.73094237, 64.67144178, -82.39080631, 45.6210511)


_CHUNK = 32

_DN = (((2,), (0,)), ((), ()))


def _body(pts_ref, lab_ref, g_ref, w0_ref, w1_ref, w2_ref, out_ref):
    g = g_ref[0]
    w0 = w0_ref[0]
    w1 = w1_ref[0]
    w2 = w2_ref[0]
    sx = 2.0 / IMG_W
    sy = 2.0 / IMG_H
    gx = g[0:1, :]
    gy = g[1:2, :]
    g3 = jnp.concatenate(
        [gx * sx, gy * sy,
         (0.5 * sx - 1.0) * gx + (0.5 * sy - 1.0) * gy], axis=0)
    e3 = jnp.concatenate(
        [w0 - w1 - w2,
         4.0 * w1 - 3.0 * w0 - w2,
         w0 - 2.0 * w1 + w2], axis=0)
    bb = out_ref.shape[0]

    def step(i, carry):
        sl = pl.ds(i * _CHUNK, _CHUNK)
        pts = pts_ref[sl, :, :]
        labf = lab_ref[sl, :, :]
        ones = jnp.ones_like(labf)
        p3 = jnp.concatenate([pts, ones], axis=2)
        l3 = jnp.concatenate([ones, labf, labf * labf], axis=2)
        u = jax.lax.dot_general(
            p3, g3, _DN, precision=jax.lax.Precision.HIGHEST,
            preferred_element_type=jnp.float32)
        delta = jax.lax.dot_general(
            l3, e3, _DN, precision=jax.lax.Precision.HIGHEST,
            preferred_element_type=jnp.float32)

        r = u - jnp.floor(u + 0.5)
        r2 = r * r
        s = r * (_S0 + r2 * (_S1 + r2 * (_S2 + r2 * _S3)))
        co = _C0 + r2 * (_C1 + r2 * (_C2 + r2 * (_C3 + r2 * _C4)))

        out_ref[sl, :, 0:128] = s + delta[:, :, 0:128]
        out_ref[sl, :, 128:256] = co + delta[:, :, 128:256]
        return carry

    jax.lax.fori_loop(0, bb // _CHUNK, step, 0)


@functools.partial(jax.jit, static_argnames=("bb",))
def _run(points, labf3, g3, w03, w13, w23, bb=256):
    b, n, _ = points.shape
    grid = b // bb
    return pl.pallas_call(
        _body,
        grid=(grid,),
        compiler_params=pltpu.CompilerParams(
            vmem_limit_bytes=100 * 1024 * 1024),
        in_specs=[
            pl.BlockSpec((bb, n, 2), lambda i: (i, 0, 0)),
            pl.BlockSpec((bb, n, 1), lambda i: (i, 0, 0)),
            pl.BlockSpec((1, 2, 128), lambda i: (0, 0, 0)),
            pl.BlockSpec((1, 1, 256), lambda i: (0, 0, 0)),
            pl.BlockSpec((1, 1, 256), lambda i: (0, 0, 0)),
            pl.BlockSpec((1, 1, 256), lambda i: (0, 0, 0)),
        ],
        out_specs=pl.BlockSpec((bb, n, 256), lambda i: (i, 0, 0)),
        out_shape=jax.ShapeDtypeStruct((b, n, EMBED_DIM), jnp.float32),
    )(points, labf3, g3, w03, w13, w23)


def kernel(points, labels, pad, pe_gaussian, w0, w1, w2):
    labf3 = labels.astype(jnp.float32)[:, :, None]
    g3 = pe_gaussian[None]
    w03 = w0[None]
    w13 = w1[None]
    w23 = w2[None]
    return _run(points, labf3, g3, w03, w13, w23)
